# baseline (speedup 1.0000x reference)
import functools

import jax
import jax.numpy as jnp
from jax import lax
from jax.experimental import pallas as pl
from jax.experimental.pallas import tpu as pltpu
from jax.experimental.pallas import tpu_sc as plsc

N = 10000
B = 100
NUM_NODES = 100
E = 320000
HH = 128

NC = 2
NS = 16
NW = NC * NS

CH = 64
E_PAD = 327680
K = E_PAD // (NW * CH)
ROWS2D = E_PAD // CH

NP_R = N + 112
RPT = NP_R // NS
NP_H = 10240
HPT = NP_H // NS
NBUF = 4
GP = 16

_mesh = plsc.VectorSubcoreMesh(core_axis_name="c", subcore_axis_name="s")


@functools.partial(
    pl.kernel,
    out_type=jax.ShapeDtypeStruct((2 * 2 * NP_H,), jnp.float32),
    mesh=_mesh,
    scratch_types=[
        pltpu.VMEM((CH,), jnp.float32),
        pltpu.VMEM((HPT,), jnp.float32),
        pltpu.VMEM((K, CH), jnp.int32),
        pltpu.VMEM((K, CH), jnp.int32),
        pltpu.VMEM_SHARED((NP_H,), jnp.float32),
        pltpu.VMEM_SHARED((NP_H,), jnp.float32),
    ],
)
def _sc_degrees(src_hbm, dst_hbm, out_hbm, ones_v, zbuf_v, srcb_v, dstb_v,
                accd_sh, accs_sh):
    c = lax.axis_index("c")
    s = lax.axis_index("s")
    wid = s * NC + c

    one16 = jnp.ones((16,), jnp.float32)
    zero16 = jnp.zeros((16,), jnp.float32)
    for j in range(CH // 16):
        ones_v[pl.ds(j * 16, 16)] = one16
    for j in range(HPT // 16):
        zbuf_v[pl.ds(j * 16, 16)] = zero16

    pltpu.sync_copy(zbuf_v, accd_sh.at[pl.ds(s * HPT, HPT)])
    pltpu.sync_copy(zbuf_v, accs_sh.at[pl.ds(s * HPT, HPT)])
    plsc.subcore_barrier()

    pltpu.sync_copy(src_hbm.at[pl.ds(wid * K, K)], srcb_v)
    pltpu.sync_copy(dst_hbm.at[pl.ds(wid * K, K)], dstb_v)

    def body(k, carry):
        pltpu.sync_copy(ones_v, accd_sh.at[dstb_v.at[k]], add=True)
        pltpu.sync_copy(ones_v, accs_sh.at[srcb_v.at[k]], add=True)
        return carry

    lax.fori_loop(0, K, body, 0)
    plsc.subcore_barrier()

    pltpu.sync_copy(accd_sh.at[pl.ds(s * HPT, HPT)],
                    out_hbm.at[pl.ds((c * 2 + 0) * NP_H + s * HPT, HPT)])
    pltpu.sync_copy(accs_sh.at[pl.ds(s * HPT, HPT)],
                    out_hbm.at[pl.ds((c * 2 + 1) * NP_H + s * HPT, HPT)])


@functools.partial(
    pl.kernel,
    out_type=jax.ShapeDtypeStruct((2 * NP_R, HH), jnp.float32),
    mesh=_mesh,
    scratch_types=[
        pltpu.VMEM((CH, HH), jnp.float32),
        pltpu.VMEM((CH, HH), jnp.float32),
        pltpu.VMEM((CH, HH), jnp.float32),
        pltpu.VMEM((CH, HH), jnp.float32),
        pltpu.VMEM((GP, CH), jnp.int32),
        pltpu.VMEM((GP, CH), jnp.int32),
        pltpu.VMEM_SHARED((NP_R, HH), jnp.float32),
        pltpu.SemaphoreType.DMA,
        pltpu.SemaphoreType.DMA,
        pltpu.SemaphoreType.DMA,
        pltpu.SemaphoreType.DMA,
    ],
)
def _sc_segsum(g_hbm, src_hbm, dst_hbm, out_hbm, rows0_v, rows1_v, rows2_v,
               rows3_v, srcb_v, dstb_v, acc_sh, sg0, sg1, sg2, sg3):
    rows_v = (rows0_v, rows1_v, rows2_v, rows3_v)
    sg = (sg0, sg1, sg2, sg3)
    c = lax.axis_index("c")
    s = lax.axis_index("s")
    wid = s * NC + c

    zero16 = jnp.zeros((16,), jnp.float32)

    def zrow(r, carry):
        for j in range(HH // 16):
            rows_v[0][r, pl.ds(j * 16, 16)] = zero16
        return carry

    lax.fori_loop(0, CH, zrow, 0)

    base = s * RPT
    for j in range(RPT // CH):
        pltpu.sync_copy(rows_v[0], acc_sh.at[pl.ds(base + j * CH, CH)])
    rem = RPT % CH
    pltpu.sync_copy(rows_v[0].at[pl.ds(0, rem)],
                    acc_sh.at[pl.ds(base + (RPT // CH) * CH, rem)])
    plsc.subcore_barrier()

    def body(gi, carry):
        e0 = wid * K + gi * GP
        pltpu.sync_copy(src_hbm.at[pl.ds(e0, GP)], srcb_v)
        pltpu.sync_copy(dst_hbm.at[pl.ds(e0, GP)], dstb_v)
        for t in range(GP // NBUF):
            descs = [
                pltpu.async_copy(g_hbm.at[srcb_v.at[t * NBUF + j]],
                                 rows_v[j], sg[j])
                for j in range(NBUF)
            ]
            for j in range(NBUF):
                descs[j].wait()
                pltpu.sync_copy(rows_v[j],
                                acc_sh.at[dstb_v.at[t * NBUF + j]], add=True)
        return carry

    lax.fori_loop(0, K // GP, body, 0)
    plsc.subcore_barrier()

    pltpu.sync_copy(acc_sh.at[pl.ds(base, RPT)],
                    out_hbm.at[pl.ds(c * NP_R + base, RPT)])


def _dinv_col(degc):
    return lax.rsqrt(degc[0] + degc[1] + 1.0)


def _tc_pre_body(x_ref, wp1_ref, bp1_ref, wp2_ref, bp2_ref, w0_ref, deg_ref,
                 pre_ref, g0_ref):
    x = x_ref[...]
    pre = jnp.maximum(x @ wp1_ref[...] + bp1_ref[...], 0.0) @ wp2_ref[...]
    pre = pre + bp2_ref[...]
    pre_ref[...] = pre
    dinv = _dinv_col(deg_ref[...])
    g = dinv * (pre @ w0_ref[...])
    g0_ref[...] = jnp.concatenate([g, jnp.zeros((NP_R - N, HH), jnp.float32)],
                                  axis=0)


def _tc_mid_body(s_ref, g_ref, b_ref, gam_ref, bet_ref, wn_ref, deg_ref,
                 gn_ref):
    S = s_ref[...]
    g = g_ref[...]
    dinv = _dinv_col(deg_ref[...])
    t = dinv * (S[:N] + S[NP_R:NP_R + N] + g[:N]) + b_ref[...]
    m = jnp.mean(t, axis=0)
    v = jnp.mean((t - m) ** 2, axis=0)
    h = jnp.maximum((t - m) * lax.rsqrt(v + 1e-5) * gam_ref[...] + bet_ref[...],
                    0.0)
    gn = dinv * (h @ wn_ref[...])
    gn_ref[...] = jnp.concatenate([gn, jnp.zeros((NP_R - N, HH), jnp.float32)],
                                  axis=0)


def _tc_final_body(s_ref, g_ref, b_ref, pre_ref, wq1_ref, bq1_ref, wq2_ref,
                   bq2_ref, xcol_ref, cap_ref, prev_ref, deg_ref, degm_ref,
                   out_ref):
    S = s_ref[...]
    g = g_ref[...]
    dinv = _dinv_col(deg_ref[...])
    t = dinv * (S[:N] + S[NP_R:NP_R + N] + g[:N]) + b_ref[...]
    h = jnp.maximum(t + pre_ref[...], 0.0)
    h = jnp.maximum(h @ wq1_ref[...] + bq1_ref[...], 0.0) @ wq2_ref[...]
    h = h + bq2_ref[...]

    prev = prev_ref[...]
    prev_flat = jnp.arange(B, dtype=jnp.int32) * NUM_NODES + prev
    node_iota = lax.broadcasted_iota(jnp.int32, (B, N), 1)
    onehot = (node_iota == prev_flat[:, None]).astype(jnp.float32)
    emb = onehot @ h
    hb = h.reshape(B, NUM_NODES, HH)
    logits = jnp.sum(hb * emb[:, None, :], axis=-1)

    degm = degm_ref[...]
    incnt = degm[0, 0] + degm[0, 1] + degm[1, 0] + degm[1, 1]
    one = jnp.float32(1.0)
    zero = jnp.float32(0.0)
    rs = jnp.maximum(jnp.where(incnt > 0.0, one, zero),
                     jnp.where(xcol_ref[...] > cap_ref[...][:, None], one,
                               zero))
    col = lax.broadcasted_iota(jnp.int32, (B, NUM_NODES), 1)
    rs = jnp.where(col == 0, jnp.where(prev[:, None] == 0, one, zero), rs)
    out_ref[...] = jnp.where(rs > 0.5, jnp.float32(-1e10), logits)


def kernel(x, edge_index, vehicle_cap, prev_node, Wp1, bp1, Wp2, bp2, convW,
           convB, bnG, bnB, Wq1, bq1, Wq2, bq2):
    src = edge_index[0]
    dst = edge_index[1]
    pad = N + (jnp.arange(E_PAD - E, dtype=jnp.int32) % 16)
    src2d = jnp.concatenate([src, pad]).reshape(ROWS2D, CH)
    dst2d = jnp.concatenate([dst, pad]).reshape(ROWS2D, CH)

    deg = _sc_degrees(src2d, dst2d).reshape(2, 2, NP_H)
    degc = deg[:, 0, :N, None]
    degm = deg[:, :, :N].reshape(2, 2, B, NUM_NODES)

    pre, g0 = pl.pallas_call(
        _tc_pre_body,
        out_shape=(
            jax.ShapeDtypeStruct((N, HH), jnp.float32),
            jax.ShapeDtypeStruct((NP_R, HH), jnp.float32),
        ),
    )(x, Wp1, bp1, Wp2, bp2, convW[0], degc)

    mid = pl.pallas_call(
        _tc_mid_body,
        out_shape=jax.ShapeDtypeStruct((NP_R, HH), jnp.float32),
    )

    S0 = _sc_segsum(g0, src2d, dst2d)
    g1 = mid(S0, g0, convB[0], bnG[0], bnB[0], convW[1], degc)
    S1 = _sc_segsum(g1, src2d, dst2d)
    g2 = mid(S1, g1, convB[1], bnG[1], bnB[1], convW[2], degc)
    S2 = _sc_segsum(g2, src2d, dst2d)

    xcol = x[:, 2].reshape(B, NUM_NODES)
    out = pl.pallas_call(
        _tc_final_body,
        out_shape=jax.ShapeDtypeStruct((B, NUM_NODES), jnp.float32),
    )(S2, g2, convB[2], pre, Wq1, bq1, Wq2, bq2, xcol, vehicle_cap,
      prev_node, degc, degm)
    return out


# speedup vs baseline: 1.0337x; 1.0337x over previous
import functools

import jax
import jax.numpy as jnp
from jax import lax
from jax.experimental import pallas as pl
from jax.experimental.pallas import tpu as pltpu
from jax.experimental.pallas import tpu_sc as plsc

N = 10000
B = 100
NUM_NODES = 100
E = 320000
HH = 128

NC = 2
NS = 16
NW = NC * NS

CH = 128
E_PAD = 327680
K = E_PAD // (NW * CH)
ROWS2D = E_PAD // CH

NP_R = N + 112
RPT = NP_R // NS
NP_H = 10240
HPT = NP_H // NS
NBUF = 2
GP = 8

_mesh = plsc.VectorSubcoreMesh(core_axis_name="c", subcore_axis_name="s")


@functools.partial(
    pl.kernel,
    out_type=jax.ShapeDtypeStruct((2 * 2 * NP_H,), jnp.float32),
    mesh=_mesh,
    scratch_types=[
        pltpu.VMEM((CH,), jnp.float32),
        pltpu.VMEM((HPT,), jnp.float32),
        pltpu.VMEM((K, CH), jnp.int32),
        pltpu.VMEM((K, CH), jnp.int32),
        pltpu.VMEM_SHARED((NP_H,), jnp.float32),
        pltpu.VMEM_SHARED((NP_H,), jnp.float32),
    ],
)
def _sc_degrees(src_hbm, dst_hbm, out_hbm, ones_v, zbuf_v, srcb_v, dstb_v,
                accd_sh, accs_sh):
    c = lax.axis_index("c")
    s = lax.axis_index("s")
    wid = s * NC + c

    one16 = jnp.ones((16,), jnp.float32)
    zero16 = jnp.zeros((16,), jnp.float32)
    for j in range(CH // 16):
        ones_v[pl.ds(j * 16, 16)] = one16
    for j in range(HPT // 16):
        zbuf_v[pl.ds(j * 16, 16)] = zero16

    pltpu.sync_copy(zbuf_v, accd_sh.at[pl.ds(s * HPT, HPT)])
    pltpu.sync_copy(zbuf_v, accs_sh.at[pl.ds(s * HPT, HPT)])
    plsc.subcore_barrier()

    pltpu.sync_copy(src_hbm.at[pl.ds(wid * K, K)], srcb_v)
    pltpu.sync_copy(dst_hbm.at[pl.ds(wid * K, K)], dstb_v)

    def body(k, carry):
        pltpu.sync_copy(ones_v, accd_sh.at[dstb_v.at[k]], add=True)
        pltpu.sync_copy(ones_v, accs_sh.at[srcb_v.at[k]], add=True)
        return carry

    lax.fori_loop(0, K, body, 0)
    plsc.subcore_barrier()

    pltpu.sync_copy(accd_sh.at[pl.ds(s * HPT, HPT)],
                    out_hbm.at[pl.ds((c * 2 + 0) * NP_H + s * HPT, HPT)])
    pltpu.sync_copy(accs_sh.at[pl.ds(s * HPT, HPT)],
                    out_hbm.at[pl.ds((c * 2 + 1) * NP_H + s * HPT, HPT)])


@functools.partial(
    pl.kernel,
    out_type=jax.ShapeDtypeStruct((2 * NP_R, HH), jnp.float32),
    mesh=_mesh,
    scratch_types=[
        pltpu.VMEM((CH, HH), jnp.float32),
        pltpu.VMEM((CH, HH), jnp.float32),
        pltpu.VMEM((GP, CH), jnp.int32),
        pltpu.VMEM((GP, CH), jnp.int32),
        pltpu.VMEM_SHARED((NP_R, HH), jnp.float32),
        pltpu.SemaphoreType.DMA,
        pltpu.SemaphoreType.DMA,
    ],
)
def _sc_segsum(g_hbm, src_hbm, dst_hbm, out_hbm, rows0_v, rows1_v, srcb_v,
               dstb_v, acc_sh, sg0, sg1):
    rows_v = (rows0_v, rows1_v)
    sg = (sg0, sg1)
    c = lax.axis_index("c")
    s = lax.axis_index("s")
    wid = s * NC + c

    zero16 = jnp.zeros((16,), jnp.float32)

    def zrow(r, carry):
        for j in range(HH // 16):
            rows_v[0][r, pl.ds(j * 16, 16)] = zero16
        return carry

    lax.fori_loop(0, CH, zrow, 0)

    base = s * RPT
    for j in range(RPT // CH):
        pltpu.sync_copy(rows_v[0], acc_sh.at[pl.ds(base + j * CH, CH)])
    rem = RPT % CH
    pltpu.sync_copy(rows_v[0].at[pl.ds(0, rem)],
                    acc_sh.at[pl.ds(base + (RPT // CH) * CH, rem)])
    plsc.subcore_barrier()

    def body(gi, carry):
        e0 = wid * K + gi * GP
        pltpu.sync_copy(src_hbm.at[pl.ds(e0, GP)], srcb_v)
        pltpu.sync_copy(dst_hbm.at[pl.ds(e0, GP)], dstb_v)
        for t in range(GP // NBUF):
            descs = [
                pltpu.async_copy(g_hbm.at[srcb_v.at[t * NBUF + j]],
                                 rows_v[j], sg[j])
                for j in range(NBUF)
            ]
            for j in range(NBUF):
                descs[j].wait()
                pltpu.sync_copy(rows_v[j],
                                acc_sh.at[dstb_v.at[t * NBUF + j]], add=True)
        return carry

    lax.fori_loop(0, K // GP, body, 0)
    plsc.subcore_barrier()

    pltpu.sync_copy(acc_sh.at[pl.ds(base, RPT)],
                    out_hbm.at[pl.ds(c * NP_R + base, RPT)])


def _dinv_col(degc):
    return lax.rsqrt(degc[0] + degc[1] + 1.0)


def _tc_pre_body(x_ref, wp1_ref, bp1_ref, wp2_ref, bp2_ref, w0_ref, deg_ref,
                 pre_ref, g0_ref):
    x = x_ref[...]
    pre = jnp.maximum(x @ wp1_ref[...] + bp1_ref[...], 0.0) @ wp2_ref[...]
    pre = pre + bp2_ref[...]
    pre_ref[...] = pre
    dinv = _dinv_col(deg_ref[...])
    g = dinv * (pre @ w0_ref[...])
    g0_ref[...] = jnp.concatenate([g, jnp.zeros((NP_R - N, HH), jnp.float32)],
                                  axis=0)


def _tc_mid_body(s_ref, g_ref, b_ref, gam_ref, bet_ref, wn_ref, deg_ref,
                 gn_ref):
    S = s_ref[...]
    g = g_ref[...]
    dinv = _dinv_col(deg_ref[...])
    t = dinv * (S[:N] + S[NP_R:NP_R + N] + g[:N]) + b_ref[...]
    m = jnp.mean(t, axis=0)
    v = jnp.mean((t - m) ** 2, axis=0)
    h = jnp.maximum((t - m) * lax.rsqrt(v + 1e-5) * gam_ref[...] + bet_ref[...],
                    0.0)
    gn = dinv * (h @ wn_ref[...])
    gn_ref[...] = jnp.concatenate([gn, jnp.zeros((NP_R - N, HH), jnp.float32)],
                                  axis=0)


def _tc_final_body(s_ref, g_ref, b_ref, pre_ref, wq1_ref, bq1_ref, wq2_ref,
                   bq2_ref, xcol_ref, cap_ref, prev_ref, deg_ref, degm_ref,
                   out_ref):
    S = s_ref[...]
    g = g_ref[...]
    dinv = _dinv_col(deg_ref[...])
    t = dinv * (S[:N] + S[NP_R:NP_R + N] + g[:N]) + b_ref[...]
    h = jnp.maximum(t + pre_ref[...], 0.0)
    h = jnp.maximum(h @ wq1_ref[...] + bq1_ref[...], 0.0) @ wq2_ref[...]
    h = h + bq2_ref[...]

    prev = prev_ref[...]
    prev_flat = jnp.arange(B, dtype=jnp.int32) * NUM_NODES + prev
    node_iota = lax.broadcasted_iota(jnp.int32, (B, N), 1)
    onehot = (node_iota == prev_flat[:, None]).astype(jnp.float32)
    emb = onehot @ h
    hb = h.reshape(B, NUM_NODES, HH)
    logits = jnp.sum(hb * emb[:, None, :], axis=-1)

    degm = degm_ref[...]
    incnt = degm[0, 0] + degm[0, 1] + degm[1, 0] + degm[1, 1]
    one = jnp.float32(1.0)
    zero = jnp.float32(0.0)
    rs = jnp.maximum(jnp.where(incnt > 0.0, one, zero),
                     jnp.where(xcol_ref[...] > cap_ref[...][:, None], one,
                               zero))
    col = lax.broadcasted_iota(jnp.int32, (B, NUM_NODES), 1)
    rs = jnp.where(col == 0, jnp.where(prev[:, None] == 0, one, zero), rs)
    out_ref[...] = jnp.where(rs > 0.5, jnp.float32(-1e10), logits)


def kernel(x, edge_index, vehicle_cap, prev_node, Wp1, bp1, Wp2, bp2, convW,
           convB, bnG, bnB, Wq1, bq1, Wq2, bq2):
    src = edge_index[0]
    dst = edge_index[1]
    pad = N + (jnp.arange(E_PAD - E, dtype=jnp.int32) % 16)
    src2d = jnp.concatenate([src, pad]).reshape(ROWS2D, CH)
    dst2d = jnp.concatenate([dst, pad]).reshape(ROWS2D, CH)

    deg = _sc_degrees(src2d, dst2d).reshape(2, 2, NP_H)
    degc = deg[:, 0, :N, None]
    degm = deg[:, :, :N].reshape(2, 2, B, NUM_NODES)

    pre, g0 = pl.pallas_call(
        _tc_pre_body,
        out_shape=(
            jax.ShapeDtypeStruct((N, HH), jnp.float32),
            jax.ShapeDtypeStruct((NP_R, HH), jnp.float32),
        ),
    )(x, Wp1, bp1, Wp2, bp2, convW[0], degc)

    mid = pl.pallas_call(
        _tc_mid_body,
        out_shape=jax.ShapeDtypeStruct((NP_R, HH), jnp.float32),
    )

    S0 = _sc_segsum(g0, src2d, dst2d)
    g1 = mid(S0, g0, convB[0], bnG[0], bnB[0], convW[1], degc)
    S1 = _sc_segsum(g1, src2d, dst2d)
    g2 = mid(S1, g1, convB[1], bnG[1], bnB[1], convW[2], degc)
    S2 = _sc_segsum(g2, src2d, dst2d)

    xcol = x[:, 2].reshape(B, NUM_NODES)
    out = pl.pallas_call(
        _tc_final_body,
        out_shape=jax.ShapeDtypeStruct((B, NUM_NODES), jnp.float32),
    )(S2, g2, convB[2], pre, Wq1, bq1, Wq2, bq2, xcol, vehicle_cap,
      prev_node, degc, degm)
    return out


# speedup vs baseline: 1.4023x; 1.3565x over previous
import functools

import jax
import jax.numpy as jnp
from jax import lax
from jax.experimental import pallas as pl
from jax.experimental.pallas import tpu as pltpu
from jax.experimental.pallas import tpu_sc as plsc

N = 10000
B = 100
NUM_NODES = 100
E = 320000
HH = 128

NC = 2
NS = 16
NW = NC * NS

CH = 128
E_PAD = 327680
K = E_PAD // (NW * CH)
ROWS2D = E_PAD // CH

NP_R = N + 112
RPT = NP_R // NS
NP_H = 10240
HPT = NP_H // NS
GP = 40

_mesh = plsc.VectorSubcoreMesh(core_axis_name="c", subcore_axis_name="s")


@functools.partial(
    pl.kernel,
    out_type=jax.ShapeDtypeStruct((2 * 2 * NP_H,), jnp.float32),
    mesh=_mesh,
    scratch_types=[
        pltpu.VMEM((CH,), jnp.float32),
        pltpu.VMEM((HPT,), jnp.float32),
        pltpu.VMEM((2 * K, CH), jnp.int32),
        pltpu.VMEM_SHARED((NP_H,), jnp.float32),
        pltpu.VMEM_SHARED((NP_H,), jnp.float32),
    ],
)
def _sc_degrees(sd_hbm, out_hbm, ones_v, zbuf_v, sdb_v, accd_sh, accs_sh):
    c = lax.axis_index("c")
    s = lax.axis_index("s")
    wid = s * NC + c

    one16 = jnp.ones((16,), jnp.float32)
    zero16 = jnp.zeros((16,), jnp.float32)
    for j in range(CH // 16):
        ones_v[pl.ds(j * 16, 16)] = one16
    for j in range(HPT // 16):
        zbuf_v[pl.ds(j * 16, 16)] = zero16

    pltpu.sync_copy(zbuf_v, accd_sh.at[pl.ds(s * HPT, HPT)])
    pltpu.sync_copy(zbuf_v, accs_sh.at[pl.ds(s * HPT, HPT)])
    plsc.subcore_barrier()

    pltpu.sync_copy(sd_hbm.at[pl.ds(2 * wid * K, 2 * K)], sdb_v)

    def body(k, carry):
        pltpu.sync_copy(ones_v, accs_sh.at[sdb_v.at[2 * k]], add=True)
        pltpu.sync_copy(ones_v, accd_sh.at[sdb_v.at[2 * k + 1]], add=True)
        return carry

    lax.fori_loop(0, K, body, 0)
    plsc.subcore_barrier()

    pltpu.sync_copy(accd_sh.at[pl.ds(s * HPT, HPT)],
                    out_hbm.at[pl.ds((c * 2 + 0) * NP_H + s * HPT, HPT)])
    pltpu.sync_copy(accs_sh.at[pl.ds(s * HPT, HPT)],
                    out_hbm.at[pl.ds((c * 2 + 1) * NP_H + s * HPT, HPT)])


@functools.partial(
    pl.kernel,
    out_type=jax.ShapeDtypeStruct((2 * NP_R, HH), jnp.float32),
    mesh=_mesh,
    scratch_types=[
        pltpu.VMEM((CH, HH), jnp.float32),
        pltpu.VMEM((CH, HH), jnp.float32),
        pltpu.VMEM((2 * GP, CH), jnp.int32),
        pltpu.VMEM_SHARED((NP_R, HH), jnp.float32),
        pltpu.SemaphoreType.DMA,
        pltpu.SemaphoreType.DMA,
    ],
)
def _sc_segsum(g_hbm, sd_hbm, out_hbm, rows0_v, rows1_v, idx_v, acc_sh,
               sg0, sg1):
    rows_v = (rows0_v, rows1_v)
    sg = (sg0, sg1)
    c = lax.axis_index("c")
    s = lax.axis_index("s")
    wid = s * NC + c

    zero16 = jnp.zeros((16,), jnp.float32)

    def zrow(r, carry):
        for j in range(HH // 16):
            rows_v[0][r, pl.ds(j * 16, 16)] = zero16
        return carry

    lax.fori_loop(0, CH, zrow, 0)

    base = s * RPT
    for j in range(RPT // CH):
        pltpu.sync_copy(rows_v[0], acc_sh.at[pl.ds(base + j * CH, CH)])
    rem = RPT % CH
    pltpu.sync_copy(rows_v[0].at[pl.ds(0, rem)],
                    acc_sh.at[pl.ds(base + (RPT // CH) * CH, rem)])
    plsc.subcore_barrier()

    def proc(t):
        j = t % 2
        pltpu.async_copy(g_hbm.at[idx_v.at[2 * t]], rows_v[j], sg[j])

    def cons(t):
        j = t % 2
        pltpu.make_async_copy(g_hbm.at[pl.ds(0, CH)], rows_v[j],
                              sg[j]).wait()
        pltpu.sync_copy(rows_v[j], acc_sh.at[idx_v.at[2 * t + 1]], add=True)

    def body(gi, carry):
        row0 = 2 * (wid * K + gi * GP)
        pltpu.sync_copy(sd_hbm.at[pl.ds(row0, 2 * GP)], idx_v)
        proc(0)
        proc(1)
        for t in range(GP - 2):
            cons(t)
            proc(t + 2)
        cons(GP - 2)
        cons(GP - 1)
        return carry

    lax.fori_loop(0, K // GP, body, 0)
    plsc.subcore_barrier()

    pltpu.sync_copy(acc_sh.at[pl.ds(base, RPT)],
                    out_hbm.at[pl.ds(c * NP_R + base, RPT)])


def _dinv_col(degc):
    return lax.rsqrt(degc[0] + degc[1] + 1.0)


def _tc_pre_body(x_ref, wp1_ref, bp1_ref, wp2_ref, bp2_ref, w0_ref, deg_ref,
                 pre_ref, g0_ref):
    x = x_ref[...]
    pre = jnp.maximum(x @ wp1_ref[...] + bp1_ref[...], 0.0) @ wp2_ref[...]
    pre = pre + bp2_ref[...]
    pre_ref[...] = pre
    dinv = _dinv_col(deg_ref[...])
    g = dinv * (pre @ w0_ref[...])
    g0_ref[...] = jnp.concatenate([g, jnp.zeros((NP_R - N, HH), jnp.float32)],
                                  axis=0)


def _tc_mid_body(s_ref, g_ref, b_ref, gam_ref, bet_ref, wn_ref, deg_ref,
                 gn_ref):
    S = s_ref[...]
    g = g_ref[...]
    dinv = _dinv_col(deg_ref[...])
    t = dinv * (S[:N] + S[NP_R:NP_R + N] + g[:N]) + b_ref[...]
    m = jnp.mean(t, axis=0)
    v = jnp.mean((t - m) ** 2, axis=0)
    h = jnp.maximum((t - m) * lax.rsqrt(v + 1e-5) * gam_ref[...] + bet_ref[...],
                    0.0)
    gn = dinv * (h @ wn_ref[...])
    gn_ref[...] = jnp.concatenate([gn, jnp.zeros((NP_R - N, HH), jnp.float32)],
                                  axis=0)


def _tc_final_body(s_ref, g_ref, b_ref, pre_ref, wq1_ref, bq1_ref, wq2_ref,
                   bq2_ref, xcol_ref, cap_ref, prev_ref, deg_ref, degm_ref,
                   out_ref):
    S = s_ref[...]
    g = g_ref[...]
    dinv = _dinv_col(deg_ref[...])
    t = dinv * (S[:N] + S[NP_R:NP_R + N] + g[:N]) + b_ref[...]
    h = jnp.maximum(t + pre_ref[...], 0.0)
    h = jnp.maximum(h @ wq1_ref[...] + bq1_ref[...], 0.0) @ wq2_ref[...]
    h = h + bq2_ref[...]

    prev = prev_ref[...]
    prev_flat = jnp.arange(B, dtype=jnp.int32) * NUM_NODES + prev
    node_iota = lax.broadcasted_iota(jnp.int32, (B, N), 1)
    onehot = (node_iota == prev_flat[:, None]).astype(jnp.float32)
    emb = onehot @ h
    hb = h.reshape(B, NUM_NODES, HH)
    logits = jnp.sum(hb * emb[:, None, :], axis=-1)

    degm = degm_ref[...]
    incnt = degm[0, 0] + degm[0, 1] + degm[1, 0] + degm[1, 1]
    one = jnp.float32(1.0)
    zero = jnp.float32(0.0)
    rs = jnp.maximum(jnp.where(incnt > 0.0, one, zero),
                     jnp.where(xcol_ref[...] > cap_ref[...][:, None], one,
                               zero))
    col = lax.broadcasted_iota(jnp.int32, (B, NUM_NODES), 1)
    rs = jnp.where(col == 0, jnp.where(prev[:, None] == 0, one, zero), rs)
    out_ref[...] = jnp.where(rs > 0.5, jnp.float32(-1e10), logits)


def kernel(x, edge_index, vehicle_cap, prev_node, Wp1, bp1, Wp2, bp2, convW,
           convB, bnG, bnB, Wq1, bq1, Wq2, bq2):
    src = edge_index[0]
    dst = edge_index[1]
    pad = N + (jnp.arange(E_PAD - E, dtype=jnp.int32) % 16)
    src2d = jnp.concatenate([src, pad]).reshape(ROWS2D, CH)
    dst2d = jnp.concatenate([dst, pad]).reshape(ROWS2D, CH)
    sd2d = jnp.stack([src2d, dst2d], axis=1).reshape(2 * ROWS2D, CH)

    deg = _sc_degrees(sd2d).reshape(2, 2, NP_H)
    degc = deg[:, 0, :N, None]
    degm = deg[:, :, :N].reshape(2, 2, B, NUM_NODES)

    pre, g0 = pl.pallas_call(
        _tc_pre_body,
        out_shape=(
            jax.ShapeDtypeStruct((N, HH), jnp.float32),
            jax.ShapeDtypeStruct((NP_R, HH), jnp.float32),
        ),
    )(x, Wp1, bp1, Wp2, bp2, convW[0], degc)

    mid = pl.pallas_call(
        _tc_mid_body,
        out_shape=jax.ShapeDtypeStruct((NP_R, HH), jnp.float32),
    )

    S0 = _sc_segsum(g0, sd2d)
    g1 = mid(S0, g0, convB[0], bnG[0], bnB[0], convW[1], degc)
    S1 = _sc_segsum(g1, sd2d)
    g2 = mid(S1, g1, convB[1], bnG[1], bnB[1], convW[2], degc)
    S2 = _sc_segsum(g2, sd2d)

    xcol = x[:, 2].reshape(B, NUM_NODES)
    out = pl.pallas_call(
        _tc_final_body,
        out_shape=jax.ShapeDtypeStruct((B, NUM_NODES), jnp.float32),
    )(S2, g2, convB[2], pre, Wq1, bq1, Wq2, bq2, xcol, vehicle_cap,
      prev_node, degc, degm)
    return out


# speedup vs baseline: 1.4214x; 1.0136x over previous
import functools

import jax
import jax.numpy as jnp
from jax import lax
from jax.experimental import pallas as pl
from jax.experimental.pallas import tpu as pltpu
from jax.experimental.pallas import tpu_sc as plsc

N = 10000
B = 100
NUM_NODES = 100
E = 320000
HH = 128

NC = 2
NS = 16
NW = NC * NS

CH = 128
E_PAD = 327680
K = E_PAD // (NW * CH)
ROWS2D = E_PAD // CH

NP_R = N + 112
RPT = NP_R // NS
NP_H = 10240
HPT = NP_H // NS
GP = 40

_mesh = plsc.VectorSubcoreMesh(core_axis_name="c", subcore_axis_name="s")


@functools.partial(
    pl.kernel,
    out_type=jax.ShapeDtypeStruct((2 * 2 * NP_H,), jnp.float32),
    mesh=_mesh,
    scratch_types=[
        pltpu.VMEM((CH,), jnp.float32),
        pltpu.VMEM((HPT,), jnp.float32),
        pltpu.VMEM((2 * K, CH), jnp.int32),
        pltpu.VMEM_SHARED((NP_H,), jnp.float32),
        pltpu.VMEM_SHARED((NP_H,), jnp.float32),
    ],
)
def _sc_degrees(sd_hbm, out_hbm, ones_v, zbuf_v, sdb_v, accd_sh, accs_sh):
    c = lax.axis_index("c")
    s = lax.axis_index("s")
    wid = s * NC + c

    one16 = jnp.ones((16,), jnp.float32)
    zero16 = jnp.zeros((16,), jnp.float32)
    for j in range(CH // 16):
        ones_v[pl.ds(j * 16, 16)] = one16
    for j in range(HPT // 16):
        zbuf_v[pl.ds(j * 16, 16)] = zero16

    pltpu.sync_copy(zbuf_v, accd_sh.at[pl.ds(s * HPT, HPT)])
    pltpu.sync_copy(zbuf_v, accs_sh.at[pl.ds(s * HPT, HPT)])
    plsc.subcore_barrier()

    pltpu.sync_copy(sd_hbm.at[pl.ds(2 * wid * K, 2 * K)], sdb_v)

    def body(k, carry):
        pltpu.sync_copy(ones_v, accs_sh.at[sdb_v.at[2 * k]], add=True)
        pltpu.sync_copy(ones_v, accd_sh.at[sdb_v.at[2 * k + 1]], add=True)
        return carry

    lax.fori_loop(0, K, body, 0)
    plsc.subcore_barrier()

    pltpu.sync_copy(accd_sh.at[pl.ds(s * HPT, HPT)],
                    out_hbm.at[pl.ds((c * 2 + 0) * NP_H + s * HPT, HPT)])
    pltpu.sync_copy(accs_sh.at[pl.ds(s * HPT, HPT)],
                    out_hbm.at[pl.ds((c * 2 + 1) * NP_H + s * HPT, HPT)])


@functools.partial(
    pl.kernel,
    out_type=jax.ShapeDtypeStruct((2 * NP_R, HH), jnp.float32),
    mesh=_mesh,
    scratch_types=[
        pltpu.VMEM((CH, HH), jnp.float32),
        pltpu.VMEM((CH, HH), jnp.float32),
        pltpu.VMEM((2 * GP, CH), jnp.int32),
        pltpu.VMEM_SHARED((NP_R, HH), jnp.float32),
        pltpu.SemaphoreType.DMA,
        pltpu.SemaphoreType.DMA,
    ],
)
def _sc_segsum(g_hbm, sd_hbm, out_hbm, rows0_v, rows1_v, idx_v, acc_sh,
               sg0, sg1):
    rows_v = (rows0_v, rows1_v)
    sg = (sg0, sg1)
    c = lax.axis_index("c")
    s = lax.axis_index("s")
    wid = s * NC + c

    zero16 = jnp.zeros((16,), jnp.float32)

    def zrow(r, carry):
        for j in range(HH // 16):
            rows_v[0][r, pl.ds(j * 16, 16)] = zero16
        return carry

    lax.fori_loop(0, CH, zrow, 0)

    base = s * RPT
    for j in range(RPT // CH):
        pltpu.sync_copy(rows_v[0], acc_sh.at[pl.ds(base + j * CH, CH)])
    rem = RPT % CH
    pltpu.sync_copy(rows_v[0].at[pl.ds(0, rem)],
                    acc_sh.at[pl.ds(base + (RPT // CH) * CH, rem)])
    plsc.subcore_barrier()

    def proc(t):
        j = t % 2
        pltpu.async_copy(g_hbm.at[idx_v.at[2 * t]], rows_v[j], sg[j])

    def cons(t):
        j = t % 2
        pltpu.make_async_copy(g_hbm.at[pl.ds(0, CH)], rows_v[j],
                              sg[j]).wait()
        pltpu.sync_copy(rows_v[j], acc_sh.at[idx_v.at[2 * t + 1]], add=True)

    def body(gi, carry):
        row0 = 2 * (wid * K + gi * GP)
        pltpu.sync_copy(sd_hbm.at[pl.ds(row0, 2 * GP)], idx_v)
        proc(0)
        proc(1)
        for t in range(GP - 2):
            cons(t)
            proc(t + 2)
        cons(GP - 2)
        cons(GP - 1)
        return carry

    lax.fori_loop(0, K // GP, body, 0)
    plsc.subcore_barrier()

    pltpu.sync_copy(acc_sh.at[pl.ds(base, RPT)],
                    out_hbm.at[pl.ds(c * NP_R + base, RPT)])


def _dinv_col(degc):
    return lax.rsqrt(degc[0] + degc[1] + 1.0)


def _tc_mlp_body(x_ref, wp1_ref, bp1_ref, wp2_ref, bp2_ref, w0_ref,
                 pre_ref, hw0_ref):
    x = x_ref[...]
    pre = jnp.maximum(x @ wp1_ref[...] + bp1_ref[...], 0.0) @ wp2_ref[...]
    pre = pre + bp2_ref[...]
    pre_ref[...] = pre
    hw0_ref[...] = pre @ w0_ref[...]


def _tc_scale_body(hw_ref, deg_ref, g0_ref):
    dinv = _dinv_col(deg_ref[...])
    g = dinv * hw_ref[...]
    g0_ref[...] = jnp.concatenate([g, jnp.zeros((NP_R - N, HH), jnp.float32)],
                                  axis=0)


def _tc_mid_body(s_ref, g_ref, b_ref, gam_ref, bet_ref, wn_ref, deg_ref,
                 gn_ref):
    S = s_ref[...]
    g = g_ref[...]
    dinv = _dinv_col(deg_ref[...])
    t = dinv * (S[:N] + S[NP_R:NP_R + N] + g[:N]) + b_ref[...]
    m = jnp.mean(t, axis=0)
    v = jnp.mean((t - m) ** 2, axis=0)
    h = jnp.maximum((t - m) * lax.rsqrt(v + 1e-5) * gam_ref[...] + bet_ref[...],
                    0.0)
    gn = dinv * (h @ wn_ref[...])
    gn_ref[...] = jnp.concatenate([gn, jnp.zeros((NP_R - N, HH), jnp.float32)],
                                  axis=0)


def _tc_final_body(s_ref, g_ref, b_ref, pre_ref, wq1_ref, bq1_ref, wq2_ref,
                   bq2_ref, xcol_ref, cap_ref, prev_ref, deg_ref, degm_ref,
                   out_ref):
    S = s_ref[...]
    g = g_ref[...]
    dinv = _dinv_col(deg_ref[...])
    t = dinv * (S[:N] + S[NP_R:NP_R + N] + g[:N]) + b_ref[...]
    h = jnp.maximum(t + pre_ref[...], 0.0)
    h = jnp.maximum(h @ wq1_ref[...] + bq1_ref[...], 0.0) @ wq2_ref[...]
    h = h + bq2_ref[...]

    prev = prev_ref[...]
    prev_flat = jnp.arange(B, dtype=jnp.int32) * NUM_NODES + prev
    node_iota = lax.broadcasted_iota(jnp.int32, (B, N), 1)
    onehot = (node_iota == prev_flat[:, None]).astype(jnp.float32)
    emb = onehot @ h
    hb = h.reshape(B, NUM_NODES, HH)
    logits = jnp.sum(hb * emb[:, None, :], axis=-1)

    degm = degm_ref[...]
    incnt = degm[0, 0] + degm[0, 1] + degm[1, 0] + degm[1, 1]
    one = jnp.float32(1.0)
    zero = jnp.float32(0.0)
    rs = jnp.maximum(jnp.where(incnt > 0.0, one, zero),
                     jnp.where(xcol_ref[...] > cap_ref[...][:, None], one,
                               zero))
    col = lax.broadcasted_iota(jnp.int32, (B, NUM_NODES), 1)
    rs = jnp.where(col == 0, jnp.where(prev[:, None] == 0, one, zero), rs)
    out_ref[...] = jnp.where(rs > 0.5, jnp.float32(-1e10), logits)


def kernel(x, edge_index, vehicle_cap, prev_node, Wp1, bp1, Wp2, bp2, convW,
           convB, bnG, bnB, Wq1, bq1, Wq2, bq2):
    src = edge_index[0]
    dst = edge_index[1]
    pad = N + (jnp.arange(E_PAD - E, dtype=jnp.int32) % 16)
    src2d = jnp.concatenate([src, pad]).reshape(ROWS2D, CH)
    dst2d = jnp.concatenate([dst, pad]).reshape(ROWS2D, CH)
    sd2d = jnp.stack([src2d, dst2d], axis=1).reshape(2 * ROWS2D, CH)

    deg = _sc_degrees(sd2d).reshape(2, 2, NP_H)
    degc = deg[:, 0, :N, None]
    degm = deg[:, :, :N].reshape(2, 2, B, NUM_NODES)

    pre, hw0 = pl.pallas_call(
        _tc_mlp_body,
        out_shape=(
            jax.ShapeDtypeStruct((N, HH), jnp.float32),
            jax.ShapeDtypeStruct((N, HH), jnp.float32),
        ),
    )(x, Wp1, bp1, Wp2, bp2, convW[0])

    g0 = pl.pallas_call(
        _tc_scale_body,
        out_shape=jax.ShapeDtypeStruct((NP_R, HH), jnp.float32),
    )(hw0, degc)

    mid = pl.pallas_call(
        _tc_mid_body,
        out_shape=jax.ShapeDtypeStruct((NP_R, HH), jnp.float32),
    )

    S0 = _sc_segsum(g0, sd2d)
    g1 = mid(S0, g0, convB[0], bnG[0], bnB[0], convW[1], degc)
    S1 = _sc_segsum(g1, sd2d)
    g2 = mid(S1, g1, convB[1], bnG[1], bnB[1], convW[2], degc)
    S2 = _sc_segsum(g2, sd2d)

    xcol = x[:, 2].reshape(B, NUM_NODES)
    out = pl.pallas_call(
        _tc_final_body,
        out_shape=jax.ShapeDtypeStruct((B, NUM_NODES), jnp.float32),
    )(S2, g2, convB[2], pre, Wq1, bq1, Wq2, bq2, xcol, vehicle_cap,
      prev_node, degc, degm)
    return out


# speedup vs baseline: 1.4320x; 1.0075x over previous
import functools

import jax
import jax.numpy as jnp
from jax import lax
from jax.experimental import pallas as pl
from jax.experimental.pallas import tpu as pltpu
from jax.experimental.pallas import tpu_sc as plsc

N = 10000
B = 100
NUM_NODES = 100
E = 320000
HH = 128

NC = 2
NS = 16
NW = NC * NS

CH = 128
E_PAD = 327680
K = E_PAD // (NW * CH)
ROWS2D = E_PAD // CH

NP_R = N + 112
RPT = NP_R // NS
NP_H = 10240
HPT = NP_H // NS
GP = 40

_mesh = plsc.VectorSubcoreMesh(core_axis_name="c", subcore_axis_name="s")


@functools.partial(
    pl.kernel,
    out_type=jax.ShapeDtypeStruct((2 * 2 * NP_H,), jnp.float32),
    mesh=_mesh,
    scratch_types=[
        pltpu.VMEM((CH,), jnp.float32),
        pltpu.VMEM((HPT,), jnp.float32),
        pltpu.VMEM((2 * K, CH), jnp.int32),
        pltpu.VMEM_SHARED((NP_H,), jnp.float32),
        pltpu.VMEM_SHARED((NP_H,), jnp.float32),
    ],
)
def _sc_degrees(sd_hbm, out_hbm, ones_v, zbuf_v, sdb_v, accd_sh, accs_sh):
    c = lax.axis_index("c")
    s = lax.axis_index("s")
    wid = s * NC + c

    one16 = jnp.ones((16,), jnp.float32)
    zero16 = jnp.zeros((16,), jnp.float32)
    for j in range(CH // 16):
        ones_v[pl.ds(j * 16, 16)] = one16
    for j in range(HPT // 16):
        zbuf_v[pl.ds(j * 16, 16)] = zero16

    pltpu.sync_copy(zbuf_v, accd_sh.at[pl.ds(s * HPT, HPT)])
    pltpu.sync_copy(zbuf_v, accs_sh.at[pl.ds(s * HPT, HPT)])
    plsc.subcore_barrier()

    pltpu.sync_copy(sd_hbm.at[pl.ds(2 * wid * K, 2 * K)], sdb_v)

    def body(k, carry):
        pltpu.sync_copy(ones_v, accs_sh.at[sdb_v.at[2 * k]], add=True)
        pltpu.sync_copy(ones_v, accd_sh.at[sdb_v.at[2 * k + 1]], add=True)
        return carry

    lax.fori_loop(0, K, body, 0)
    plsc.subcore_barrier()

    pltpu.sync_copy(accd_sh.at[pl.ds(s * HPT, HPT)],
                    out_hbm.at[pl.ds((c * 2 + 0) * NP_H + s * HPT, HPT)])
    pltpu.sync_copy(accs_sh.at[pl.ds(s * HPT, HPT)],
                    out_hbm.at[pl.ds((c * 2 + 1) * NP_H + s * HPT, HPT)])


@functools.partial(
    pl.kernel,
    out_type=jax.ShapeDtypeStruct((2 * NP_R, HH), jnp.float32),
    mesh=_mesh,
    scratch_types=[
        pltpu.VMEM((CH, HH), jnp.float32),
        pltpu.VMEM((CH, HH), jnp.float32),
        pltpu.VMEM((2 * GP, CH), jnp.int32),
        pltpu.VMEM_SHARED((NP_R, HH), jnp.float32),
        pltpu.SemaphoreType.DMA,
        pltpu.SemaphoreType.DMA,
    ],
)
def _sc_segsum(g_hbm, sd_hbm, out_hbm, rows0_v, rows1_v, idx_v, acc_sh,
               sg0, sg1):
    rows_v = (rows0_v, rows1_v)
    sg = (sg0, sg1)
    c = lax.axis_index("c")
    s = lax.axis_index("s")
    wid = s * NC + c

    zero16 = jnp.zeros((16,), jnp.float32)

    def proc(t):
        j = t % 2
        pltpu.async_copy(g_hbm.at[idx_v.at[2 * t]], rows_v[j], sg[j])

    def cons(t):
        j = t % 2
        pltpu.make_async_copy(g_hbm.at[pl.ds(0, CH)], rows_v[j],
                              sg[j]).wait()
        pltpu.sync_copy(rows_v[j], acc_sh.at[idx_v.at[2 * t + 1]], add=True)

    def tail(carry):
        for t in range(GP - 2):
            cons(t)
            proc(t + 2)
        cons(GP - 2)
        cons(GP - 1)
        return carry

    pltpu.sync_copy(sd_hbm.at[pl.ds(2 * wid * K, 2 * GP)], idx_v)
    proc(0)

    def zrow(r, carry):
        for j in range(HH // 16):
            rows_v[1][r, pl.ds(j * 16, 16)] = zero16
        return carry

    lax.fori_loop(0, CH, zrow, 0)

    base = s * RPT
    for j in range(RPT // CH):
        pltpu.sync_copy(rows_v[1], acc_sh.at[pl.ds(base + j * CH, CH)])
    rem = RPT % CH
    pltpu.sync_copy(rows_v[1].at[pl.ds(0, rem)],
                    acc_sh.at[pl.ds(base + (RPT // CH) * CH, rem)])
    plsc.subcore_barrier()
    proc(1)
    tail(0)

    def body(gi, carry):
        row0 = 2 * (wid * K + gi * GP)
        pltpu.sync_copy(sd_hbm.at[pl.ds(row0, 2 * GP)], idx_v)
        proc(0)
        proc(1)
        return tail(carry)

    lax.fori_loop(1, K // GP, body, 0)
    plsc.subcore_barrier()

    pltpu.sync_copy(acc_sh.at[pl.ds(base, RPT)],
                    out_hbm.at[pl.ds(c * NP_R + base, RPT)])


def _dinv_col(degc):
    return lax.rsqrt(degc[0] + degc[1] + 1.0)


def _tc_mlp_body(x_ref, wp1_ref, bp1_ref, wp2_ref, bp2_ref, w0_ref,
                 pre_ref, hw0_ref):
    x = x_ref[...]
    pre = jnp.maximum(x @ wp1_ref[...] + bp1_ref[...], 0.0) @ wp2_ref[...]
    pre = pre + bp2_ref[...]
    pre_ref[...] = pre
    hw0_ref[...] = pre @ w0_ref[...]


def _tc_scale_body(hw_ref, deg_ref, g0_ref):
    dinv = _dinv_col(deg_ref[...])
    g = dinv * hw_ref[...]
    g0_ref[...] = jnp.concatenate([g, jnp.zeros((NP_R - N, HH), jnp.float32)],
                                  axis=0)


def _tc_mid_body(s_ref, g_ref, b_ref, gam_ref, bet_ref, wn_ref, deg_ref,
                 gn_ref):
    S = s_ref[...]
    g = g_ref[...]
    dinv = _dinv_col(deg_ref[...])
    t = dinv * (S[:N] + S[NP_R:NP_R + N] + g[:N]) + b_ref[...]
    m = jnp.mean(t, axis=0)
    v = jnp.mean((t - m) ** 2, axis=0)
    h = jnp.maximum((t - m) * lax.rsqrt(v + 1e-5) * gam_ref[...] + bet_ref[...],
                    0.0)
    gn = dinv * (h @ wn_ref[...])
    gn_ref[...] = jnp.concatenate([gn, jnp.zeros((NP_R - N, HH), jnp.float32)],
                                  axis=0)


def _tc_final_body(s_ref, g_ref, b_ref, pre_ref, wq1_ref, bq1_ref, wq2_ref,
                   bq2_ref, xcol_ref, cap_ref, prev_ref, deg_ref, degm_ref,
                   out_ref):
    S = s_ref[...]
    g = g_ref[...]
    dinv = _dinv_col(deg_ref[...])
    t = dinv * (S[:N] + S[NP_R:NP_R + N] + g[:N]) + b_ref[...]
    h = jnp.maximum(t + pre_ref[...], 0.0)
    h = jnp.maximum(h @ wq1_ref[...] + bq1_ref[...], 0.0) @ wq2_ref[...]
    h = h + bq2_ref[...]

    prev = prev_ref[...]
    prev_flat = jnp.arange(B, dtype=jnp.int32) * NUM_NODES + prev
    node_iota = lax.broadcasted_iota(jnp.int32, (B, N), 1)
    onehot = (node_iota == prev_flat[:, None]).astype(jnp.float32)
    emb = onehot @ h
    hb = h.reshape(B, NUM_NODES, HH)
    logits = jnp.sum(hb * emb[:, None, :], axis=-1)

    degm = degm_ref[...]
    incnt = degm[0, 0] + degm[0, 1] + degm[1, 0] + degm[1, 1]
    one = jnp.float32(1.0)
    zero = jnp.float32(0.0)
    rs = jnp.maximum(jnp.where(incnt > 0.0, one, zero),
                     jnp.where(xcol_ref[...] > cap_ref[...][:, None], one,
                               zero))
    col = lax.broadcasted_iota(jnp.int32, (B, NUM_NODES), 1)
    rs = jnp.where(col == 0, jnp.where(prev[:, None] == 0, one, zero), rs)
    out_ref[...] = jnp.where(rs > 0.5, jnp.float32(-1e10), logits)


def kernel(x, edge_index, vehicle_cap, prev_node, Wp1, bp1, Wp2, bp2, convW,
           convB, bnG, bnB, Wq1, bq1, Wq2, bq2):
    src = edge_index[0]
    dst = edge_index[1]
    pad = N + (jnp.arange(E_PAD - E, dtype=jnp.int32) % 16)
    src2d = jnp.concatenate([src, pad]).reshape(ROWS2D, CH)
    dst2d = jnp.concatenate([dst, pad]).reshape(ROWS2D, CH)
    sd2d = jnp.stack([src2d, dst2d], axis=1).reshape(2 * ROWS2D, CH)

    deg = _sc_degrees(sd2d).reshape(2, 2, NP_H)
    degc = deg[:, 0, :N, None]
    degm = deg[:, :, :N].reshape(2, 2, B, NUM_NODES)

    pre, hw0 = pl.pallas_call(
        _tc_mlp_body,
        out_shape=(
            jax.ShapeDtypeStruct((N, HH), jnp.float32),
            jax.ShapeDtypeStruct((N, HH), jnp.float32),
        ),
    )(x, Wp1, bp1, Wp2, bp2, convW[0])

    g0 = pl.pallas_call(
        _tc_scale_body,
        out_shape=jax.ShapeDtypeStruct((NP_R, HH), jnp.float32),
    )(hw0, degc)

    mid = pl.pallas_call(
        _tc_mid_body,
        out_shape=jax.ShapeDtypeStruct((NP_R, HH), jnp.float32),
    )

    S0 = _sc_segsum(g0, sd2d)
    g1 = mid(S0, g0, convB[0], bnG[0], bnB[0], convW[1], degc)
    S1 = _sc_segsum(g1, sd2d)
    g2 = mid(S1, g1, convB[1], bnG[1], bnB[1], convW[2], degc)
    S2 = _sc_segsum(g2, sd2d)

    xcol = x[:, 2].reshape(B, NUM_NODES)
    out = pl.pallas_call(
        _tc_final_body,
        out_shape=jax.ShapeDtypeStruct((B, NUM_NODES), jnp.float32),
    )(S2, g2, convB[2], pre, Wq1, bq1, Wq2, bq2, xcol, vehicle_cap,
      prev_node, degc, degm)
    return out


# speedup vs baseline: 1.4585x; 1.0185x over previous
import functools

import jax
import jax.numpy as jnp
from jax import lax
from jax.experimental import pallas as pl
from jax.experimental.pallas import tpu as pltpu
from jax.experimental.pallas import tpu_sc as plsc

N = 10000
B = 100
NUM_NODES = 100
E = 320000
HH = 128

NC = 2
NS = 16
NW = NC * NS

CH = 128
E_PAD = 327680
K = E_PAD // (NW * CH)
ROWS2D = E_PAD // CH

NP_R = N + 112
RPT = NP_R // NS
NP_H = 10240
HPT = NP_H // NS
GP = 40

_mesh = plsc.VectorSubcoreMesh(core_axis_name="c", subcore_axis_name="s")


@functools.partial(
    pl.kernel,
    out_type=jax.ShapeDtypeStruct((2 * 2 * NP_H,), jnp.float32),
    mesh=_mesh,
    scratch_types=[
        pltpu.VMEM((CH,), jnp.float32),
        pltpu.VMEM((HPT,), jnp.float32),
        pltpu.VMEM((2 * K, CH), jnp.int32),
        pltpu.VMEM_SHARED((NP_H,), jnp.float32),
        pltpu.VMEM_SHARED((NP_H,), jnp.float32),
        pltpu.SemaphoreType.DMA,
        pltpu.SemaphoreType.DMA,
    ],
)
def _sc_degrees(sd_hbm, out_hbm, ones_v, zbuf_v, sdb_v, accd_sh, accs_sh,
                sem_s, sem_d):
    c = lax.axis_index("c")
    s = lax.axis_index("s")
    wid = s * NC + c

    one16 = jnp.ones((16,), jnp.float32)
    zero16 = jnp.zeros((16,), jnp.float32)
    for j in range(CH // 16):
        ones_v[pl.ds(j * 16, 16)] = one16
    for j in range(HPT // 16):
        zbuf_v[pl.ds(j * 16, 16)] = zero16

    pltpu.sync_copy(zbuf_v, accd_sh.at[pl.ds(s * HPT, HPT)])
    pltpu.sync_copy(zbuf_v, accs_sh.at[pl.ds(s * HPT, HPT)])
    plsc.subcore_barrier()

    pltpu.sync_copy(sd_hbm.at[pl.ds(2 * wid * K, 2 * K)], sdb_v)

    def body(i, carry):
        for u in range(4):
            k = i * 4 + u
            pltpu.async_copy(ones_v, accs_sh.at[sdb_v.at[2 * k]], sem_s,
                             add=True)
            pltpu.async_copy(ones_v, accd_sh.at[sdb_v.at[2 * k + 1]], sem_d,
                             add=True)
        for u in range(4):
            pltpu.make_async_copy(out_hbm.at[pl.ds(0, CH)], ones_v,
                                  sem_s).wait()
            pltpu.make_async_copy(out_hbm.at[pl.ds(0, CH)], ones_v,
                                  sem_d).wait()
        return carry

    lax.fori_loop(0, K // 4, body, 0)
    plsc.subcore_barrier()

    pltpu.sync_copy(accd_sh.at[pl.ds(s * HPT, HPT)],
                    out_hbm.at[pl.ds((c * 2 + 0) * NP_H + s * HPT, HPT)])
    pltpu.sync_copy(accs_sh.at[pl.ds(s * HPT, HPT)],
                    out_hbm.at[pl.ds((c * 2 + 1) * NP_H + s * HPT, HPT)])


@functools.partial(
    pl.kernel,
    out_type=jax.ShapeDtypeStruct((2 * NP_R, HH), jnp.float32),
    mesh=_mesh,
    scratch_types=[
        pltpu.VMEM((CH, HH), jnp.float32),
        pltpu.VMEM((CH, HH), jnp.float32),
        pltpu.VMEM((2 * GP, CH), jnp.int32),
        pltpu.VMEM_SHARED((NP_R, HH), jnp.float32),
        pltpu.SemaphoreType.DMA,
        pltpu.SemaphoreType.DMA,
    ],
)
def _sc_segsum(g_hbm, sd_hbm, out_hbm, rows0_v, rows1_v, idx_v, acc_sh,
               sg0, sg1):
    rows_v = (rows0_v, rows1_v)
    sg = (sg0, sg1)
    c = lax.axis_index("c")
    s = lax.axis_index("s")
    wid = s * NC + c

    zero16 = jnp.zeros((16,), jnp.float32)

    def proc(t):
        j = t % 2
        pltpu.async_copy(g_hbm.at[idx_v.at[2 * t]], rows_v[j], sg[j])

    def cons(t):
        j = t % 2
        pltpu.make_async_copy(g_hbm.at[pl.ds(0, CH)], rows_v[j],
                              sg[j]).wait()
        pltpu.sync_copy(rows_v[j], acc_sh.at[idx_v.at[2 * t + 1]], add=True)

    def tail(carry):
        for t in range(GP - 2):
            cons(t)
            proc(t + 2)
        cons(GP - 2)
        cons(GP - 1)
        return carry

    pltpu.sync_copy(sd_hbm.at[pl.ds(2 * wid * K, 2 * GP)], idx_v)
    proc(0)

    def zrow(r, carry):
        for j in range(HH // 16):
            rows_v[1][r, pl.ds(j * 16, 16)] = zero16
        return carry

    lax.fori_loop(0, CH, zrow, 0)

    base = s * RPT
    for j in range(RPT // CH):
        pltpu.sync_copy(rows_v[1], acc_sh.at[pl.ds(base + j * CH, CH)])
    rem = RPT % CH
    pltpu.sync_copy(rows_v[1].at[pl.ds(0, rem)],
                    acc_sh.at[pl.ds(base + (RPT // CH) * CH, rem)])
    plsc.subcore_barrier()
    proc(1)
    tail(0)

    def body(gi, carry):
        row0 = 2 * (wid * K + gi * GP)
        pltpu.sync_copy(sd_hbm.at[pl.ds(row0, 2 * GP)], idx_v)
        proc(0)
        proc(1)
        return tail(carry)

    lax.fori_loop(1, K // GP, body, 0)
    plsc.subcore_barrier()

    pltpu.sync_copy(acc_sh.at[pl.ds(base, RPT)],
                    out_hbm.at[pl.ds(c * NP_R + base, RPT)])


def _dinv_col(degc):
    return lax.rsqrt(degc[0] + degc[1] + 1.0)


def _tc_mlp_body(x_ref, wp1_ref, bp1_ref, wp2_ref, bp2_ref, w0_ref,
                 pre_ref, hw0_ref):
    x = x_ref[...]
    pre = jnp.maximum(x @ wp1_ref[...] + bp1_ref[...], 0.0) @ wp2_ref[...]
    pre = pre + bp2_ref[...]
    pre_ref[...] = pre
    hw0_ref[...] = pre @ w0_ref[...]


def _tc_scale_body(hw_ref, deg_ref, g0_ref):
    dinv = _dinv_col(deg_ref[...])
    g = dinv * hw_ref[...]
    g0_ref[...] = jnp.concatenate([g, jnp.zeros((NP_R - N, HH), jnp.float32)],
                                  axis=0)


def _tc_mid_body(s_ref, g_ref, b_ref, gam_ref, bet_ref, wn_ref, deg_ref,
                 gn_ref):
    S = s_ref[...]
    g = g_ref[...]
    dinv = _dinv_col(deg_ref[...])
    t = dinv * (S[:N] + S[NP_R:NP_R + N] + g[:N]) + b_ref[...]
    m = jnp.mean(t, axis=0)
    v = jnp.mean((t - m) ** 2, axis=0)
    h = jnp.maximum((t - m) * lax.rsqrt(v + 1e-5) * gam_ref[...] + bet_ref[...],
                    0.0)
    gn = dinv * (h @ wn_ref[...])
    gn_ref[...] = jnp.concatenate([gn, jnp.zeros((NP_R - N, HH), jnp.float32)],
                                  axis=0)


def _tc_final_body(s_ref, g_ref, b_ref, pre_ref, wq1_ref, bq1_ref, wq2_ref,
                   bq2_ref, xcol_ref, cap_ref, prev_ref, deg_ref, degm_ref,
                   out_ref):
    S = s_ref[...]
    g = g_ref[...]
    dinv = _dinv_col(deg_ref[...])
    t = dinv * (S[:N] + S[NP_R:NP_R + N] + g[:N]) + b_ref[...]
    h = jnp.maximum(t + pre_ref[...], 0.0)
    h = jnp.maximum(h @ wq1_ref[...] + bq1_ref[...], 0.0) @ wq2_ref[...]
    h = h + bq2_ref[...]

    prev = prev_ref[...]
    prev_flat = jnp.arange(B, dtype=jnp.int32) * NUM_NODES + prev
    node_iota = lax.broadcasted_iota(jnp.int32, (B, N), 1)
    onehot = (node_iota == prev_flat[:, None]).astype(jnp.float32)
    emb = onehot @ h
    hb = h.reshape(B, NUM_NODES, HH)
    logits = jnp.sum(hb * emb[:, None, :], axis=-1)

    degm = degm_ref[...]
    incnt = degm[0, 0] + degm[0, 1] + degm[1, 0] + degm[1, 1]
    one = jnp.float32(1.0)
    zero = jnp.float32(0.0)
    rs = jnp.maximum(jnp.where(incnt > 0.0, one, zero),
                     jnp.where(xcol_ref[...] > cap_ref[...][:, None], one,
                               zero))
    col = lax.broadcasted_iota(jnp.int32, (B, NUM_NODES), 1)
    rs = jnp.where(col == 0, jnp.where(prev[:, None] == 0, one, zero), rs)
    out_ref[...] = jnp.where(rs > 0.5, jnp.float32(-1e10), logits)


def kernel(x, edge_index, vehicle_cap, prev_node, Wp1, bp1, Wp2, bp2, convW,
           convB, bnG, bnB, Wq1, bq1, Wq2, bq2):
    src = edge_index[0]
    dst = edge_index[1]
    pad = N + (jnp.arange(E_PAD - E, dtype=jnp.int32) % 16)
    src2d = jnp.concatenate([src, pad]).reshape(ROWS2D, CH)
    dst2d = jnp.concatenate([dst, pad]).reshape(ROWS2D, CH)
    sd2d = jnp.stack([src2d, dst2d], axis=1).reshape(2 * ROWS2D, CH)

    deg = _sc_degrees(sd2d).reshape(2, 2, NP_H)
    degc = deg[:, 0, :N, None]
    degm = deg[:, :, :N].reshape(2, 2, B, NUM_NODES)

    pre, hw0 = pl.pallas_call(
        _tc_mlp_body,
        out_shape=(
            jax.ShapeDtypeStruct((N, HH), jnp.float32),
            jax.ShapeDtypeStruct((N, HH), jnp.float32),
        ),
    )(x, Wp1, bp1, Wp2, bp2, convW[0])

    g0 = pl.pallas_call(
        _tc_scale_body,
        out_shape=jax.ShapeDtypeStruct((NP_R, HH), jnp.float32),
    )(hw0, degc)

    mid = pl.pallas_call(
        _tc_mid_body,
        out_shape=jax.ShapeDtypeStruct((NP_R, HH), jnp.float32),
    )

    S0 = _sc_segsum(g0, sd2d)
    g1 = mid(S0, g0, convB[0], bnG[0], bnB[0], convW[1], degc)
    S1 = _sc_segsum(g1, sd2d)
    g2 = mid(S1, g1, convB[1], bnG[1], bnB[1], convW[2], degc)
    S2 = _sc_segsum(g2, sd2d)

    xcol = x[:, 2].reshape(B, NUM_NODES)
    out = pl.pallas_call(
        _tc_final_body,
        out_shape=jax.ShapeDtypeStruct((B, NUM_NODES), jnp.float32),
    )(S2, g2, convB[2], pre, Wq1, bq1, Wq2, bq2, xcol, vehicle_cap,
      prev_node, degc, degm)
    return out
